# baseline (device time: 70834 ns/iter reference)
import jax
import jax.numpy as jnp
from jax import lax
from jax.experimental import pallas as pl
from jax.experimental.pallas import tpu as pltpu

N_DEV = 4
EPS = 1e-5
BLK = 1024
RING = 4
PRE = 2
LAG = 2


def kernel(x, gamma):
    m, n_local = x.shape
    n_global = n_local * N_DEV
    G = m // BLK
    tb = BLK // 128
    g2 = gamma.reshape(1, n_local)

    def body(x_hbm, g_ref, out_ref, ring, comm_ref, dma_sems,
             send_sems, recv_sems):
        i = pl.program_id(0)
        my = lax.axis_index("i")

        def in_dma(b):
            slot = b % RING if isinstance(b, int) else lax.rem(b, RING)
            return pltpu.make_async_copy(
                x_hbm.at[pl.ds(b * BLK, BLK), :],
                ring.at[slot],
                dma_sems.at[slot],
            )

        def partial_rdma(k, b):
            return pltpu.make_async_remote_copy(
                src_ref=comm_ref.at[0, b],
                dst_ref=comm_ref.at[N_DEV - k, b],
                send_sem=send_sems.at[k - 1, b],
                recv_sem=recv_sems.at[N_DEV - k, b],
                device_id=(lax.rem(my + k, N_DEV),),
                device_id_type=pl.DeviceIdType.MESH,
            )

        @pl.when(i == 0)
        def _start():
            barrier = pltpu.get_barrier_semaphore()
            for k in range(1, N_DEV):
                peer = lax.rem(my + k, N_DEV)
                pl.semaphore_signal(
                    barrier, inc=1,
                    device_id=(peer,), device_id_type=pl.DeviceIdType.MESH,
                )
            pl.semaphore_wait(barrier, N_DEV - 1)
            for b in range(min(PRE, G)):
                in_dma(b).start()

        @pl.when((i > 0) & (i + PRE - 1 < G))
        def _prefetch():
            in_dma(i + PRE - 1).start()

        @pl.when(i < G)
        def _partial():
            in_dma(i).wait()
            x3 = ring[lax.rem(i, RING)].reshape(tb, 128, n_local)
            comm_ref[0, i] = jnp.sum(x3 * x3, axis=2)
            for k in range(1, N_DEV):
                partial_rdma(k, i).start()

        @pl.when(i >= LAG)
        def _scale():
            j = i - LAG
            for k in range(1, N_DEV):
                partial_rdma(k, j).wait()
            total = (
                comm_ref[0, j] + comm_ref[1, j]
                + comm_ref[2, j] + comm_ref[3, j]
            )
            inv3 = lax.rsqrt(total / n_global + EPS).reshape(tb, 128, 1)
            x3 = ring[lax.rem(j, RING)].reshape(tb, 128, n_local)
            gw = g_ref[:, :].reshape(1, 1, n_local)
            out_ref[:, :] = (x3 * inv3 * gw).reshape(BLK, n_local)

    return pl.pallas_call(
        body,
        grid=(G + LAG,),
        out_shape=jax.ShapeDtypeStruct((m, n_local), x.dtype),
        in_specs=[
            pl.BlockSpec(memory_space=pltpu.MemorySpace.HBM),
            pl.BlockSpec((1, n_local), lambda i: (0, 0)),
        ],
        out_specs=pl.BlockSpec(
            (BLK, n_local), lambda i: (jnp.maximum(i - LAG, 0), 0)
        ),
        scratch_shapes=[
            pltpu.VMEM((RING, BLK, n_local), jnp.float32),
            pltpu.VMEM((N_DEV, G, tb, 128), jnp.float32),
            pltpu.SemaphoreType.DMA((RING,)),
            pltpu.SemaphoreType.DMA((N_DEV - 1, G)),
            pltpu.SemaphoreType.DMA((N_DEV, G)),
        ],
        compiler_params=pltpu.CompilerParams(
            collective_id=0,
            vmem_limit_bytes=60 * 1024 * 1024,
        ),
    )(x, g2)
